# baseline (speedup 1.0000x reference)
import functools

import jax
import jax.numpy as jnp
from jax import lax
from jax.experimental import pallas as pl
from jax.experimental.pallas import tpu as pltpu
from jax.experimental.pallas import tpu_sc as plsc

L = 16


def _sc_body(n_l_chunks, l_rows, b_cols, n_total_b,
             wtab_hbm, x_hbm, o1_hbm, o2_hbm, o3_hbm,
             w_v, t1_v, t2_v, t3_v,
             xa_v, xb_v, o1a_v, o2a_v, o3a_v, o1b_v, o2b_v, o3b_v,
             sia, sib, soa, sob):
    wid = lax.axis_index("s") * 2 + lax.axis_index("c")
    li = wid // (32 // n_l_chunks)
    bi = wid % (32 // n_l_chunks)
    l0 = li * l_rows
    b0 = bi * b_cols

    pltpu.sync_copy(wtab_hbm, w_v)
    t1 = w_v[10]
    for d in range(5):
        t1 = t1 + w_v[d] * w_v[5 + d]
    t2 = w_v[21]
    for d in range(5):
        t2 = t2 + w_v[11 + d] * w_v[16 + d]
    z3 = w_v[42]
    for d in range(10):
        z3 = z3 + w_v[22 + d] * w_v[32 + d]
    ones = jnp.ones((L,), jnp.float32)
    t3 = ones / (ones + jnp.exp(-z3))
    t1_v[...] = t1
    t2_v[...] = t2
    t3_v[...] = t3

    G = 8
    iters = b_cols // (L * G)

    x_b = [xa_v, xb_v]
    o_b = [[o1a_v, o2a_v, o3a_v], [o1b_v, o2b_v, o3b_v]]
    o_hbm = [o1_hbm, o2_hbm, o3_hbm]
    sin = [sia, sib]
    sout = [soa, sob]

    def compute(x_v, o1_v, o2_v, o3_v):
        def gather_group(i, _):
            start = i * (L * G)
            offs = [start + g * L for g in range(G)]
            idxs = [x_v[pl.ds(o, L)] for o in offs]
            r1 = [plsc.load_gather(t1_v, [idx]) for idx in idxs]
            r2 = [plsc.load_gather(t2_v, [idx]) for idx in idxs]
            r3 = [plsc.load_gather(t3_v, [idx]) for idx in idxs]
            for g in range(G):
                o1_v[pl.ds(offs[g], L)] = r1[g]
                o2_v[pl.ds(offs[g], L)] = r2[g]
                o3_v[pl.ds(offs[g], L)] = r3[g]
            return 0
        lax.fori_loop(0, iters, gather_group, 0)

    def seg(s):
        return (l0 + s) * n_total_b + b0

    pltpu.async_copy(x_hbm.at[pl.ds(seg(0), b_cols)], x_b[0], sin[0])

    for s in range(l_rows):
        b = s % 2
        off = seg(s)
        if s + 1 < l_rows:
            pltpu.async_copy(
                x_hbm.at[pl.ds(seg(s + 1), b_cols)], x_b[1 - b], sin[1 - b])
        pltpu.make_async_copy(
            x_hbm.at[pl.ds(off, b_cols)], x_b[b], sin[b]).wait()
        if s >= 2:
            prev = seg(s - 2)
            for k in range(3):
                pltpu.make_async_copy(
                    o_b[b][k], o_hbm[k].at[pl.ds(prev, b_cols)],
                    sout[b]).wait()
        compute(x_b[b], *o_b[b])
        for k in range(3):
            pltpu.async_copy(
                o_b[b][k], o_hbm[k].at[pl.ds(off, b_cols)], sout[b])

    for s in (l_rows - 2, l_rows - 1):
        b = s % 2
        off = seg(s)
        for k in range(3):
            pltpu.make_async_copy(
                o_b[b][k], o_hbm[k].at[pl.ds(off, b_cols)], sout[b]).wait()


def kernel(x, emb1_W, lin1_W, lin1_b, emb2_W, lin2_W, lin2_b,
           emb3_W, lin3_W, lin3_b):
    B, ncols = x.shape
    n = B * ncols

    n_l_chunks = 8
    l_rows = ncols // n_l_chunks
    b_cols = B // (32 // n_l_chunks)
    assert ncols % n_l_chunks == 0 and B % (32 // n_l_chunks) == 0

    def colpack(emb_W, lin_W, lin_b):
        d = emb_W.shape[1]
        cols = jnp.zeros((d, L), jnp.float32).at[:, : emb_W.shape[0]].set(emb_W.T)
        lw = jnp.broadcast_to(lin_W[0][:, None], (d, L))
        bias = jnp.broadcast_to(lin_b[0], (1, L))
        return jnp.concatenate([cols, lw, bias], axis=0)

    wtab = jnp.concatenate(
        [colpack(emb1_W, lin1_W, lin1_b),
         colpack(emb2_W, lin2_W, lin2_b),
         colpack(emb3_W, lin3_W, lin3_b)], axis=0)

    xt = jnp.transpose(x).reshape(n)

    mesh = plsc.VectorSubcoreMesh(core_axis_name="c", subcore_axis_name="s")
    f32 = jnp.float32
    out = pl.kernel(
        functools.partial(_sc_body, n_l_chunks, l_rows, b_cols, B),
        mesh=mesh,
        out_type=[jax.ShapeDtypeStruct((n,), f32)] * 3,
        scratch_types=[
            pltpu.VMEM((43, L), f32),
            pltpu.VMEM((L,), f32),
            pltpu.VMEM((L,), f32),
            pltpu.VMEM((L,), f32),
            pltpu.VMEM((b_cols,), jnp.int32),
            pltpu.VMEM((b_cols,), jnp.int32),
            pltpu.VMEM((b_cols,), f32),
            pltpu.VMEM((b_cols,), f32),
            pltpu.VMEM((b_cols,), f32),
            pltpu.VMEM((b_cols,), f32),
            pltpu.VMEM((b_cols,), f32),
            pltpu.VMEM((b_cols,), f32),
            pltpu.SemaphoreType.DMA,
            pltpu.SemaphoreType.DMA,
            pltpu.SemaphoreType.DMA,
            pltpu.SemaphoreType.DMA,
        ],
        compiler_params=pltpu.CompilerParams(needs_layout_passes=False),
    )(wtab, xt)

    return tuple(o.reshape(ncols, B).T[:, :, None] for o in out)


# speedup vs baseline: 1.2442x; 1.2442x over previous
import functools

import jax
import jax.numpy as jnp
from jax import lax
from jax.experimental import pallas as pl
from jax.experimental.pallas import tpu as pltpu
from jax.experimental.pallas import tpu_sc as plsc

L = 16


def _sc_body(rows_per_worker, rblk, ncols, num_cores,
             wtab_hbm, x_hbm, o1_hbm, o2_hbm, o3_hbm,
             w_v, t1_v, t2_v, t3_v,
             xa_v, xb_v, o1a_v, o2a_v, o3a_v, o1b_v, o2b_v, o3b_v,
             sia, sib, soa, sob):
    wid = lax.axis_index("s") * num_cores + lax.axis_index("c")
    base = wid * rows_per_worker

    pltpu.sync_copy(wtab_hbm, w_v)
    t1 = w_v[10]
    for d in range(5):
        t1 = t1 + w_v[d] * w_v[5 + d]
    t2 = w_v[21]
    for d in range(5):
        t2 = t2 + w_v[11 + d] * w_v[16 + d]
    z3 = w_v[42]
    for d in range(10):
        z3 = z3 + w_v[22 + d] * w_v[32 + d]
    ones = jnp.ones((L,), jnp.float32)
    t3 = ones / (ones + jnp.exp(-z3))
    t1_v[...] = t1
    t2_v[...] = t2
    t3_v[...] = t3

    nsub = rows_per_worker // rblk

    cstarts = list(range(0, ncols - L + 1, L))
    if cstarts[-1] != ncols - L:
        cstarts.append(ncols - L)

    x_b = [xa_v, xb_v]
    o_b = [[o1a_v, o2a_v, o3a_v], [o1b_v, o2b_v, o3b_v]]
    o_hbm = [o1_hbm, o2_hbm, o3_hbm]
    sin = [sia, sib]
    sout = [soa, sob]

    def compute(x_v, o1_v, o2_v, o3_v):
        def row_body(r, _):
            idxs = [x_v[r, pl.ds(c, L)] for c in cstarts]
            r1 = [plsc.load_gather(t1_v, [idx]) for idx in idxs]
            r2 = [plsc.load_gather(t2_v, [idx]) for idx in idxs]
            r3 = [plsc.load_gather(t3_v, [idx]) for idx in idxs]
            for g, c in enumerate(cstarts):
                o1_v[r, pl.ds(c, L)] = r1[g]
                o2_v[r, pl.ds(c, L)] = r2[g]
                o3_v[r, pl.ds(c, L)] = r3[g]
            return 0
        lax.fori_loop(0, rblk, row_body, 0)

    pltpu.async_copy(x_hbm.at[pl.ds(base, rblk), :], x_b[0], sin[0])

    for s in range(nsub):
        b = s % 2
        row0 = base + s * rblk
        if s + 1 < nsub:
            pltpu.async_copy(
                x_hbm.at[pl.ds(row0 + rblk, rblk), :], x_b[1 - b], sin[1 - b])
        pltpu.make_async_copy(
            x_hbm.at[pl.ds(row0, rblk), :], x_b[b], sin[b]).wait()
        if s >= 2:
            prev = row0 - 2 * rblk
            for k in range(3):
                pltpu.make_async_copy(
                    o_b[b][k], o_hbm[k].at[pl.ds(prev, rblk), :],
                    sout[b]).wait()
        compute(x_b[b], *o_b[b])
        for k in range(3):
            pltpu.async_copy(
                o_b[b][k], o_hbm[k].at[pl.ds(row0, rblk), :], sout[b])

    for s in (nsub - 2, nsub - 1):
        b = s % 2
        row0 = base + s * rblk
        for k in range(3):
            pltpu.make_async_copy(
                o_b[b][k], o_hbm[k].at[pl.ds(row0, rblk), :], sout[b]).wait()


def kernel(x, emb1_W, lin1_W, lin1_b, emb2_W, lin2_W, lin2_b,
           emb3_W, lin3_W, lin3_b):
    B, ncols = x.shape

    info = plsc.get_sparse_core_info()
    nw = info.num_cores * info.num_subcores
    rows_per_worker = B // nw
    rblk = 32
    assert rows_per_worker % rblk == 0

    def colpack(emb_W, lin_W, lin_b):
        d = emb_W.shape[1]
        cols = jnp.zeros((d, L), jnp.float32).at[:, : emb_W.shape[0]].set(emb_W.T)
        lw = jnp.broadcast_to(lin_W[0][:, None], (d, L))
        bias = jnp.broadcast_to(lin_b[0], (1, L))
        return jnp.concatenate([cols, lw, bias], axis=0)

    wtab = jnp.concatenate(
        [colpack(emb1_W, lin1_W, lin1_b),
         colpack(emb2_W, lin2_W, lin2_b),
         colpack(emb3_W, lin3_W, lin3_b)], axis=0)

    mesh = plsc.VectorSubcoreMesh(core_axis_name="c", subcore_axis_name="s")
    f32 = jnp.float32
    out = pl.kernel(
        functools.partial(_sc_body, rows_per_worker, rblk, ncols,
                          info.num_cores),
        mesh=mesh,
        out_type=[jax.ShapeDtypeStruct((B, ncols), f32)] * 3,
        scratch_types=[
            pltpu.VMEM((43, L), f32),
            pltpu.VMEM((L,), f32),
            pltpu.VMEM((L,), f32),
            pltpu.VMEM((L,), f32),
            pltpu.VMEM((rblk, ncols), jnp.int32),
            pltpu.VMEM((rblk, ncols), jnp.int32),
            pltpu.VMEM((rblk, ncols), f32),
            pltpu.VMEM((rblk, ncols), f32),
            pltpu.VMEM((rblk, ncols), f32),
            pltpu.VMEM((rblk, ncols), f32),
            pltpu.VMEM((rblk, ncols), f32),
            pltpu.VMEM((rblk, ncols), f32),
            pltpu.SemaphoreType.DMA,
            pltpu.SemaphoreType.DMA,
            pltpu.SemaphoreType.DMA,
            pltpu.SemaphoreType.DMA,
        ],
        compiler_params=pltpu.CompilerParams(
            needs_layout_passes=False, use_tc_tiling_on_sc=True),
    )(wtab, x)

    return tuple(o[:, :, None] for o in out)


# speedup vs baseline: 1.4966x; 1.2028x over previous
import functools

import jax
import jax.numpy as jnp
from jax import lax
from jax.experimental import pallas as pl
from jax.experimental.pallas import tpu as pltpu
from jax.experimental.pallas import tpu_sc as plsc

L = 16


def _sc_body(nl, nb, lblk, bblk, num_cores,
             wtab_hbm, xt_hbm, o1_hbm, o2_hbm, o3_hbm,
             w_v, t1_v, t2_v, t3_v,
             xa_v, xb_v, o1a_v, o2a_v, o3a_v, o1b_v, o2b_v, o3b_v,
             sia, sib, soa, sob):
    wid = lax.axis_index("s") * num_cores + lax.axis_index("c")

    pltpu.sync_copy(wtab_hbm, w_v)
    t1 = w_v[10]
    for d in range(5):
        t1 = t1 + w_v[d] * w_v[5 + d]
    t2 = w_v[21]
    for d in range(5):
        t2 = t2 + w_v[11 + d] * w_v[16 + d]
    z3 = w_v[42]
    for d in range(10):
        z3 = z3 + w_v[22 + d] * w_v[32 + d]
    ones = jnp.ones((L,), jnp.float32)
    t3 = ones / (ones + jnp.exp(-z3))
    t1_v[...] = t1
    t2_v[...] = t2
    t3_v[...] = t3

    nlb = nl // lblk
    nchunks = 2 * nlb

    def chunk_slice(c):
        stripe = 2 * wid + c // nlb
        l0 = (c % nlb) * lblk
        return (pl.ds(l0, lblk), pl.ds(stripe * bblk, bblk))

    x_b = [xa_v, xb_v]
    o_b = [[o1a_v, o2a_v, o3a_v], [o1b_v, o2b_v, o3b_v]]
    o_hbm = [o1_hbm, o2_hbm, o3_hbm]
    sin = [sia, sib]
    sout = [soa, sob]

    G = 16
    assert bblk == L * G

    def compute(x_v, o1_v, o2_v, o3_v):
        def row_body(r, _):
            offs = [g * L for g in range(G)]
            idxs = [x_v[r, pl.ds(o, L)] for o in offs]
            r1 = [plsc.load_gather(t1_v, [idx]) for idx in idxs]
            r2 = [plsc.load_gather(t2_v, [idx]) for idx in idxs]
            r3 = [plsc.load_gather(t3_v, [idx]) for idx in idxs]
            for g in range(G):
                o1_v[r, pl.ds(offs[g], L)] = r1[g]
                o2_v[r, pl.ds(offs[g], L)] = r2[g]
                o3_v[r, pl.ds(offs[g], L)] = r3[g]
            return 0
        lax.fori_loop(0, lblk, row_body, 0)

    pltpu.async_copy(xt_hbm.at[chunk_slice(0)], x_b[0], sin[0])

    for s in range(nchunks):
        b = s % 2
        sl = chunk_slice(s)
        if s + 1 < nchunks:
            pltpu.async_copy(xt_hbm.at[chunk_slice(s + 1)], x_b[1 - b],
                             sin[1 - b])
        pltpu.make_async_copy(xt_hbm.at[sl], x_b[b], sin[b]).wait()
        if s >= 2:
            psl = chunk_slice(s - 2)
            for k in range(3):
                pltpu.make_async_copy(
                    o_b[b][k], o_hbm[k].at[psl], sout[b]).wait()
        compute(x_b[b], *o_b[b])
        for k in range(3):
            pltpu.async_copy(o_b[b][k], o_hbm[k].at[sl], sout[b])

    for s in (nchunks - 2, nchunks - 1):
        b = s % 2
        sl = chunk_slice(s)
        for k in range(3):
            pltpu.make_async_copy(o_b[b][k], o_hbm[k].at[sl], sout[b]).wait()


def kernel(x, emb1_W, lin1_W, lin1_b, emb2_W, lin2_W, lin2_b,
           emb3_W, lin3_W, lin3_b):
    B, ncols = x.shape

    info = plsc.get_sparse_core_info()
    nw = info.num_cores * info.num_subcores
    bblk = 256
    lblk = 40
    assert B % (2 * nw * bblk) == 0 or B == 2 * nw * bblk
    assert ncols % lblk == 0

    def colpack(emb_W, lin_W, lin_b):
        d = emb_W.shape[1]
        cols = jnp.zeros((d, L), jnp.float32).at[:, : emb_W.shape[0]].set(emb_W.T)
        lw = jnp.broadcast_to(lin_W[0][:, None], (d, L))
        bias = jnp.broadcast_to(lin_b[0], (1, L))
        return jnp.concatenate([cols, lw, bias], axis=0)

    wtab = jnp.concatenate(
        [colpack(emb1_W, lin1_W, lin1_b),
         colpack(emb2_W, lin2_W, lin2_b),
         colpack(emb3_W, lin3_W, lin3_b)], axis=0)

    xt = jnp.transpose(x)

    mesh = plsc.VectorSubcoreMesh(core_axis_name="c", subcore_axis_name="s")
    f32 = jnp.float32
    out = pl.kernel(
        functools.partial(_sc_body, ncols, B, lblk, bblk, info.num_cores),
        mesh=mesh,
        out_type=[jax.ShapeDtypeStruct((ncols, B), f32)] * 3,
        scratch_types=[
            pltpu.VMEM((43, L), f32),
            pltpu.VMEM((L,), f32),
            pltpu.VMEM((L,), f32),
            pltpu.VMEM((L,), f32),
            pltpu.VMEM((lblk, bblk), jnp.int32),
            pltpu.VMEM((lblk, bblk), jnp.int32),
            pltpu.VMEM((lblk, bblk), f32),
            pltpu.VMEM((lblk, bblk), f32),
            pltpu.VMEM((lblk, bblk), f32),
            pltpu.VMEM((lblk, bblk), f32),
            pltpu.VMEM((lblk, bblk), f32),
            pltpu.VMEM((lblk, bblk), f32),
            pltpu.SemaphoreType.DMA,
            pltpu.SemaphoreType.DMA,
            pltpu.SemaphoreType.DMA,
            pltpu.SemaphoreType.DMA,
        ],
        compiler_params=pltpu.CompilerParams(
            needs_layout_passes=False, use_tc_tiling_on_sc=True),
    )(wtab, xt)

    return tuple(o.T[:, :, None] for o in out)
